# Initial kernel scaffold; baseline (speedup 1.0000x reference)
import jax
import jax.numpy as jnp
from jax.experimental import pallas as pl


def kernel(x, edge_index, W0l, b0l, W0r, g0, be0, W1l, b1l, W1r, g1, be1, Wc, bc):
    raise NotImplementedError("write your pallas kernel here")


# speedup vs baseline: 3.9123x; 3.9123x over previous
import functools

import jax
import jax.numpy as jnp
from jax import lax
from jax.experimental import pallas as pl
from jax.experimental.pallas import tpu as pltpu
from jax.experimental.pallas import tpu_sc as plsc

N = 10000
E = 320000
D = 128
C = 2

NC = 2
NS = 16
NW = NC * NS
CHUNK = 128
CPW = 79
EPAD = NW * CPW * CHUNK
ROWS_PER_TILE = 640
NP = NS * ROWS_PER_TILE
ZROWS = 8

_mesh = plsc.VectorSubcoreMesh(core_axis_name="c", subcore_axis_name="s",
                               num_cores=NC, num_subcores=NS)


def _make_sc_aggregate(with_deg):
    out_type = [jax.ShapeDtypeStruct((NC, NP, D), jnp.float32)]
    scratch = [
        pltpu.VMEM((CHUNK,), jnp.int32),
        pltpu.VMEM((CHUNK,), jnp.int32),
        pltpu.VMEM((CHUNK, D), jnp.float32),
        pltpu.VMEM((ZROWS, D), jnp.float32),
        pltpu.VMEM_SHARED((NP, D), jnp.float32),
        pltpu.SemaphoreType.DMA,
    ]
    if with_deg:
        out_type.append(jax.ShapeDtypeStruct((NC, NP, D), jnp.float32))
        scratch.append(pltpu.VMEM((CHUNK, D), jnp.float32))

    def body(x_hbm, src_hbm, dst_hbm, *rest):
        if with_deg:
            (sum_out, deg_out, srcidx, dstidx, rows, zbuf, acc, sem,
             ones) = rest
        else:
            sum_out, srcidx, dstidx, rows, zbuf, acc, sem = rest
        c = lax.axis_index("c")
        s = lax.axis_index("s")
        wid = s * NC + c
        base = s * ROWS_PER_TILE

        zero16 = jnp.zeros((16,), jnp.float32)
        for r in range(ZROWS):
            for l in range(D // 16):
                zbuf[r, pl.ds(l * 16, 16)] = zero16

        def zacc(j, _):
            pltpu.sync_copy(zbuf, acc.at[pl.ds(base + j * ZROWS, ZROWS)])
            return 0
        lax.fori_loop(0, ROWS_PER_TILE // ZROWS, zacc, 0)

        if with_deg:
            one16 = jnp.ones((16,), jnp.float32)
            for r in range(CHUNK):
                for l in range(D // 16):
                    ones[r, pl.ds(l * 16, 16)] = one16

        plsc.subcore_barrier()

        def step(j, _):
            pltpu.sync_copy(src_hbm.at[wid, j], srcidx)
            pltpu.sync_copy(dst_hbm.at[wid, j], dstidx)
            pltpu.async_copy(x_hbm.at[srcidx], rows, sem).wait()
            pltpu.sync_copy(rows, acc.at[dstidx], add=True)
            return 0
        lax.fori_loop(0, CPW, step, 0)

        plsc.subcore_barrier()

        pltpu.sync_copy(acc.at[pl.ds(base, ROWS_PER_TILE)],
                        sum_out.at[c, pl.ds(base, ROWS_PER_TILE)])

        if with_deg:
            plsc.subcore_barrier()

            def zacc2(j, _):
                pltpu.sync_copy(zbuf, acc.at[pl.ds(base + j * ZROWS, ZROWS)])
                return 0
            lax.fori_loop(0, ROWS_PER_TILE // ZROWS, zacc2, 0)

            plsc.subcore_barrier()

            def dstep(j, _):
                pltpu.sync_copy(dst_hbm.at[wid, j], dstidx)
                pltpu.sync_copy(ones, acc.at[dstidx], add=True)
                return 0
            lax.fori_loop(0, CPW, dstep, 0)

            plsc.subcore_barrier()
            pltpu.sync_copy(acc.at[pl.ds(base, ROWS_PER_TILE)],
                            deg_out.at[c, pl.ds(base, ROWS_PER_TILE)])

    return pl.kernel(body, out_type=out_type, mesh=_mesh,
                     scratch_types=scratch)


_sc_aggregate_deg = _make_sc_aggregate(True)
_sc_aggregate = _make_sc_aggregate(False)


def _tc_layer_body(head, sum_ref, deg_ref, x_ref, wl_ref, bl_ref, wr_ref,
                   g_ref, be_ref, *rest):
    if head:
        wc_ref, bc_ref, out_ref = rest
    else:
        (out_ref,) = rest
    ssum = sum_ref[0, :N, :] + sum_ref[1, :N, :]
    deg = deg_ref[0] + deg_ref[1]
    deg = jnp.maximum(deg, 1.0)
    mean = ssum / deg
    x = x_ref[...]
    dn = (((1,), (1,)), ((), ()))
    h = (lax.dot_general(mean, wl_ref[...], dn,
                         preferred_element_type=jnp.float32)
         + bl_ref[...]
         + lax.dot_general(x, wr_ref[...], dn,
                           preferred_element_type=jnp.float32))
    m = jnp.mean(h, axis=0, keepdims=True)
    hc = h - m
    v = jnp.mean(hc * hc, axis=0, keepdims=True)
    h = jnp.maximum(g_ref[...] * hc * lax.rsqrt(v + 1e-5) + be_ref[...], 0.0)
    if head:
        out_ref[...] = (lax.dot_general(h, wc_ref[...], dn,
                                        preferred_element_type=jnp.float32)
                        + bc_ref[...])
    else:
        out_ref[...] = h


def _tc_layer(ssum, degcol, x, wl, bl, wr, g, be, wc=None, bc=None):
    head = wc is not None
    args = [ssum, degcol, x, wl, bl[None, :], wr, g[None, :], be[None, :]]
    if head:
        args += [wc, bc[None, :]]
        out_shape = jax.ShapeDtypeStruct((N, C), jnp.float32)
    else:
        out_shape = jax.ShapeDtypeStruct((N, D), jnp.float32)
    return pl.pallas_call(
        functools.partial(_tc_layer_body, head),
        out_shape=out_shape,
    )(*args)


def kernel(x, edge_index, W0l, b0l, W0r, g0, be0, W1l, b1l, W1r, g1, be1,
           Wc, bc):
    src = edge_index[0]
    dst = edge_index[1]
    pad = EPAD - E
    src3 = jnp.concatenate(
        [src, jnp.zeros((pad,), jnp.int32)]).reshape(NW, CPW, CHUNK)
    dst3 = jnp.concatenate(
        [dst, jnp.full((pad,), N, jnp.int32)]).reshape(NW, CPW, CHUNK)

    sum0, degfull = _sc_aggregate_deg(x, src3, dst3)
    degcol = degfull[:, :N, 0:1]
    h0 = _tc_layer(sum0, degcol, x, W0l, b0l, W0r, g0, be0)
    (sum1,) = _sc_aggregate(h0, src3, dst3)
    return _tc_layer(sum1, degcol, h0, W1l, b1l, W1r, g1, be1, Wc, bc)
